# baseline (speedup 1.0000x reference)
import jax
import jax.numpy as jnp
from jax import lax
from jax.experimental import pallas as pl
from jax.experimental.pallas import tpu as pltpu
from jax.experimental.pallas import tpu_sc as plsc

N = 10000
E = 320000
D = 128

NC = 2
NS = 16
NW = NC * NS
EC = E // NW
RW = 80
ROWS_W = EC // RW
NP = 10240
NPT = NP // NS
PB = 2000
PBR = PB // RW
F32 = jnp.float32


def _prep_body(h_ref, w_ref, wr_ref, att_ref, loop_ref,
               h1_ref, hl_ref, s1_ref, s2_ref, wv_ref):
    h1 = lax.dot_general(h_ref[...], w_ref[...], (((1,), (1,)), ((), ())),
                         preferred_element_type=F32)
    h1_ref[...] = h1
    hl_ref[...] = jnp.dot(h1, loop_ref[...], preferred_element_type=F32)
    s1_ref[...] = lax.dot_general(h1, att_ref[:, 0:D], (((1,), (1,)), ((), ())),
                                  preferred_element_type=F32)
    s2_ref[...] = lax.dot_general(h1, att_ref[:, D:2 * D], (((1,), (1,)), ((), ())),
                                  preferred_element_type=F32)
    wv_ref[...] = lax.dot_general(att_ref[:, 2 * D:3 * D], wr_ref[...],
                                  (((1,), (0,)), ((), ())),
                                  preferred_element_type=F32)


def _prep(h, W, Wr, att_w, loop_w):
    return pl.pallas_call(
        _prep_body,
        out_shape=[
            jax.ShapeDtypeStruct((N, D), F32),
            jax.ShapeDtypeStruct((N, D), F32),
            jax.ShapeDtypeStruct((N, 1), F32),
            jax.ShapeDtypeStruct((N, 1), F32),
            jax.ShapeDtypeStruct((1, D), F32),
        ],
    )(h, W, Wr, att_w, loop_w)


_BE = 4000


def _er_body(x_ref, wv_ref, er_ref):
    er_ref[...] = lax.dot_general(x_ref[...], wv_ref[...],
                                  (((1,), (1,)), ((), ())),
                                  preferred_element_type=F32)


def _er(r_h, wv):
    return pl.pallas_call(
        _er_body,
        grid=(E // _BE,),
        in_specs=[
            pl.BlockSpec((_BE, D), lambda i: (i, 0)),
            pl.BlockSpec((1, D), lambda i: (0, 0)),
        ],
        out_specs=pl.BlockSpec((_BE, 1), lambda i: (i, 0)),
        out_shape=jax.ShapeDtypeStruct((E, 1), F32),
    )(r_h, wv)


def _rh_body(x_ref, wr_ref, rh_ref):
    rh_ref[...] = lax.dot_general(x_ref[...], wr_ref[...],
                                  (((1,), (1,)), ((), ())),
                                  preferred_element_type=F32)


def _rh(r_h, Wr):
    return pl.pallas_call(
        _rh_body,
        grid=(E // _BE,),
        in_specs=[
            pl.BlockSpec((_BE, D), lambda i: (i, 0)),
            pl.BlockSpec((D, D), lambda i: (0, 0)),
        ],
        out_specs=pl.BlockSpec((_BE, D), lambda i: (i, 0)),
        out_shape=jax.ShapeDtypeStruct((E, D), F32),
    )(r_h, Wr)


_BN = 2000


def _epi_body(aggp_ref, hl_ref, h1_ref, den_ref, o_ref):
    h_new = aggp_ref[...] + hl_ref[...]
    sel = jnp.where(den_ref[...] > 0, h_new, h1_ref[...])
    o_ref[...] = jnp.maximum(sel, 0.0)


def _epi(aggp, hl, h1, den):
    return pl.pallas_call(
        _epi_body,
        grid=(N // _BN,),
        in_specs=[
            pl.BlockSpec((_BN, D), lambda i: (i, 0)),
            pl.BlockSpec((_BN, D), lambda i: (i, 0)),
            pl.BlockSpec((_BN, D), lambda i: (i, 0)),
            pl.BlockSpec((_BN, 1), lambda i: (i, 0)),
        ],
        out_specs=pl.BlockSpec((_BN, D), lambda i: (i, 0)),
        out_shape=jax.ShapeDtypeStruct((N, D), F32),
    )(aggp, hl, h1, den)


_SC_PARAMS = pltpu.CompilerParams(use_tc_tiling_on_sc=False,
                                  needs_layout_passes=False)


def _soft_body(src2_h, dst2_h, er_h, s1_h, s2_h,
               alpha_out, den_out,
               exo, mxv, mxm, mx_sh, den_sh):
    c = lax.axis_index("c")
    s = lax.axis_index("s")
    wid = c * NS + s
    own = wid * EC
    mir = ((1 - c) * NS + s) * EC
    zv = jnp.zeros((16,), F32)

    def bphase(s1v, s2v, srcb2, dstb2, erb, exb):
        def zden_body(i, _):
            erb[pl.ds(i * 16, 16)] = zv
            return 0
        lax.fori_loop(0, NPT // 16, zden_body, 0)
        pltpu.sync_copy(erb.at[pl.ds(0, NPT)], den_sh.at[pl.ds(s * NPT, NPT)])

        pltpu.sync_copy(s1_h, s1v)
        pltpu.sync_copy(s2_h, s2v)

        def b1_piece(base, p, store):
            off = base + p * PB
            pltpu.sync_copy(src2_h.at[pl.ds(off // RW, PBR), :], srcb2)
            pltpu.sync_copy(dst2_h.at[pl.ds(off // RW, PBR), :], dstb2)
            pltpu.sync_copy(er_h.at[pl.ds(off, PB)], erb)

            def row(j, m):
                for k in range(RW // 16):
                    sl = pl.ds(k * 16, 16)
                    z = (plsc.load_gather(s1v, [srcb2[j, sl]])
                         + plsc.load_gather(s2v, [dstb2[j, sl]])
                         + erb[pl.ds(j * RW + k * 16, 16)])
                    e = jnp.where(z >= 0, z, z * F32(0.01))
                    if store:
                        exo[pl.ds(p * PB + j * RW + k * 16, 16)] = e
                    m = jnp.maximum(m, e)
                return m
            return lax.fori_loop(0, PBR, row, jnp.full((16,), -jnp.inf, F32))

        def b1_own(p, mx):
            return jnp.maximum(mx, b1_piece(own, p, True))
        mx = lax.fori_loop(0, EC // PB, b1_own,
                           jnp.full((16,), -jnp.inf, F32))

        def b1_mir(p, mx):
            return jnp.maximum(mx, b1_piece(mir, p, False))
        mx = lax.fori_loop(0, EC // PB, b1_mir, mx)

        mxv[...] = mx
        pltpu.sync_copy(mxv, mx_sh.at[s])
        plsc.subcore_barrier()
        pltpu.sync_copy(mx_sh, mxm)
        cm = mxm[0, :]
        for i in range(1, NS):
            cm = jnp.maximum(cm, mxm[i, :])
        gmax = jnp.max(cm)

        def expo(i, _):
            sl = pl.ds(i * 16, 16)
            exo[sl] = jnp.exp(exo[sl] - gmax)
            return 0
        lax.fori_loop(0, EC // 16, expo, 0)

        def b2_own(p, _):
            off = own + p * PB
            pltpu.sync_copy(dst2_h.at[pl.ds(off // RW, PBR), :], dstb2)

            def sc2(j, __):
                pltpu.sync_copy(exo.at[pl.ds(p * PB + j * RW, RW)],
                                den_sh.at[dstb2.at[j]], add=True)
                return 0
            lax.fori_loop(0, PBR, sc2, 0)
            return 0
        lax.fori_loop(0, EC // PB, b2_own, 0)

        def b2_mir(p, _):
            off = mir + p * PB
            pltpu.sync_copy(src2_h.at[pl.ds(off // RW, PBR), :], srcb2)
            pltpu.sync_copy(dst2_h.at[pl.ds(off // RW, PBR), :], dstb2)
            pltpu.sync_copy(er_h.at[pl.ds(off, PB)], erb)

            def row(j, __):
                for k in range(RW // 16):
                    sl = pl.ds(k * 16, 16)
                    z = (plsc.load_gather(s1v, [srcb2[j, sl]])
                         + plsc.load_gather(s2v, [dstb2[j, sl]])
                         + erb[pl.ds(j * RW + k * 16, 16)])
                    e = jnp.where(z >= 0, z, z * F32(0.01))
                    exb[pl.ds(j * RW + k * 16, 16)] = jnp.exp(e - gmax)
                return 0
            lax.fori_loop(0, PBR, row, 0)

            def sc2(j, __):
                pltpu.sync_copy(exb.at[pl.ds(j * RW, RW)],
                                den_sh.at[dstb2.at[j]], add=True)
                return 0
            lax.fori_loop(0, PBR, sc2, 0)
            return 0
        lax.fori_loop(0, EC // PB, b2_mir, 0)

    pl.run_scoped(bphase,
                  pltpu.VMEM((N,), F32),
                  pltpu.VMEM((N,), F32),
                  pltpu.VMEM((PBR, RW), jnp.int32),
                  pltpu.VMEM((PBR, RW), jnp.int32),
                  pltpu.VMEM((PB,), F32),
                  pltpu.VMEM((PB,), F32))
    plsc.subcore_barrier()

    def foldphase(rdv, dstb2):
        pltpu.sync_copy(den_sh, rdv)

        def rd_body(i, _):
            sl = pl.ds(i * 16, 16)
            v = rdv[sl]
            rdv[sl] = jnp.where(v > 0, F32(1.0) / v, F32(1.0))
            return 0
        lax.fori_loop(0, NP // 16, rd_body, 0)
        pltpu.sync_copy(den_sh.at[pl.ds(s * NPT, NPT)],
                        den_out.at[c, pl.ds(s * NPT, NPT)])

        def fold_piece(p, _):
            off = own + p * PB
            pltpu.sync_copy(dst2_h.at[pl.ds(off // RW, PBR), :], dstb2)

            def row(j, __):
                for k in range(RW // 16):
                    rd = plsc.load_gather(rdv, [dstb2[j, pl.ds(k * 16, 16)]])
                    sl = pl.ds(p * PB + j * RW + k * 16, 16)
                    exo[sl] = exo[sl] * rd
                return 0
            lax.fori_loop(0, PBR, row, 0)
            return 0
        lax.fori_loop(0, EC // PB, fold_piece, 0)
        pltpu.sync_copy(exo.at[pl.ds(0, EC)], alpha_out.at[pl.ds(own, EC)])

    pl.run_scoped(foldphase,
                  pltpu.VMEM((NP,), F32),
                  pltpu.VMEM((PBR, RW), jnp.int32))


def _sc_softmax(src2, dst2, er, s1, s2):
    mesh = plsc.VectorSubcoreMesh(core_axis_name="c", subcore_axis_name="s")
    f = pl.kernel(
        _soft_body, mesh=mesh, compiler_params=_SC_PARAMS,
        out_type=[
            jax.ShapeDtypeStruct((E,), F32),
            jax.ShapeDtypeStruct((NC, NP), F32),
        ],
        scratch_types=[
            pltpu.VMEM((EC + 16,), F32),
            pltpu.VMEM((16,), F32),
            pltpu.VMEM((NS, 16), F32),
            pltpu.VMEM_SHARED((NS, 16), F32),
            pltpu.VMEM_SHARED((NP,), F32),
        ],
    )
    return f(src2, dst2, er, s1, s2)


NO = NP // NW
CAP = 12064
PC = 40


def _agg_body(src2_h, dst2_h, al_h, h1_h, rh_h, agg_out,
              src_l, ed_l, al_l, agg_l, srcb2, dstb2, alb,
              h1q3, rhq3, eidq3, gsem, rsem):
    c = lax.axis_index("c")
    s = lax.axis_index("s")
    wid = c * NS + s
    lo = wid * NO
    zv = jnp.zeros((16,), F32)
    zvi = jnp.zeros((16,), jnp.int32)
    iota = lax.iota(jnp.int32, 16)

    def scan_piece(p, cnt):
        off = p * PB
        pltpu.sync_copy(src2_h.at[pl.ds(off // RW, PBR), :], srcb2)
        pltpu.sync_copy(dst2_h.at[pl.ds(off // RW, PBR), :], dstb2)
        pltpu.sync_copy(al_h.at[pl.ds(off, PB)], alb)

        def row(j, cnt):
            for k in range(RW // 16):
                sl = pl.ds(k * 16, 16)
                dv = dstb2[j, sl]
                msk = (dv >= lo) & (dv < lo + NO)
                eid = iota + (off + j * RW + k * 16)
                ed = eid * 512 + (dv - lo)
                plsc.store_compressed(src_l.at[pl.ds(cnt, 16)],
                                      srcb2[j, sl], mask=msk)
                plsc.store_compressed(ed_l.at[pl.ds(cnt, 16)], ed, mask=msk)
                plsc.store_compressed(al_l.at[pl.ds(cnt, 16)],
                                      alb[pl.ds(j * RW + k * 16, 16)],
                                      mask=msk)
                cnt = cnt + plsc.all_reduce_population_count(msk)[0]
            return cnt
        return lax.fori_loop(0, PBR, row, cnt)

    cnt = lax.fori_loop(0, E // PB, scan_piece, jnp.int32(0))

    src_l[pl.ds(cnt, 16)] = zvi
    ed_l[pl.ds(cnt, 16)] = zvi
    al_l[pl.ds(cnt, 16)] = zv
    src_l[pl.ds(cnt + 16, 16)] = zvi
    ed_l[pl.ds(cnt + 16, 16)] = zvi
    al_l[pl.ds(cnt + 16, 16)] = zv
    src_l[pl.ds(cnt + 32, 16)] = zvi
    ed_l[pl.ds(cnt + 32, 16)] = zvi
    al_l[pl.ds(cnt + 32, 16)] = zv
    nq = (cnt + PC - 1) // PC

    def zagg(i, _):
        for kk in range(8):
            agg_l[i, pl.ds(kk * 16, 16)] = zv
        return 0
    lax.fori_loop(0, NO, zagg, 0)

    def unpack_issue(q):
        b = q % 3
        for k in range(3):
            v = ed_l[pl.ds(q * PC + k * 16, 16)]
            eidq3[b, pl.ds(k * 16, 16)] = lax.shift_right_logical(v, 9)
        pltpu.async_copy(h1_h.at[src_l.at[pl.ds(q * PC, PC)]],
                         h1q3.at[b], gsem.at[b])
        pltpu.async_copy(rh_h.at[eidq3.at[b, pl.ds(0, PC)]],
                         rhq3.at[b], rsem.at[b])

    def wait_q(q):
        b = q % 3
        pltpu.make_async_copy(h1_h.at[src_l.at[pl.ds(q * PC, PC)]],
                              h1q3.at[b], gsem.at[b]).wait()
        pltpu.make_async_copy(rh_h.at[eidq3.at[b, pl.ds(0, PC)]],
                              rhq3.at[b], rsem.at[b]).wait()

    @pl.when(nq > 0)
    def _():
        unpack_issue(0)

    @pl.when(nq > 1)
    def _():
        unpack_issue(1)

    def apiece(q, _):
        b = q % 3

        @pl.when(q + 2 < nq)
        def _():
            unpack_issue(q + 2)

        wait_q(q)

        def edge(i, __):
            v = ed_l[pl.ds(q * PC + i, 16)][0]
            dl = lax.rem(v, 512)
            a = al_l[pl.ds(q * PC + i, 16)][0]
            for kk in range(8):
                sl = pl.ds(kk * 16, 16)
                agg_l[dl, sl] = (agg_l[dl, sl]
                                 + a * (h1q3[b, i, sl] + rhq3[b, i, sl]))
            return 0
        lax.fori_loop(0, PC, edge, 0)
        return 0
    lax.fori_loop(0, nq, apiece, 0)

    pltpu.sync_copy(agg_l, agg_out.at[pl.ds(lo, NO), :])


def _sc_agg(src2, dst2, alpha, h1, rh):
    mesh = plsc.VectorSubcoreMesh(core_axis_name="c", subcore_axis_name="s")
    f = pl.kernel(
        _agg_body, mesh=mesh, compiler_params=_SC_PARAMS,
        out_type=jax.ShapeDtypeStruct((NP, D), F32),
        scratch_types=[
            pltpu.VMEM((CAP,), jnp.int32),
            pltpu.VMEM((CAP,), jnp.int32),
            pltpu.VMEM((CAP,), F32),
            pltpu.VMEM((NO, D), F32),
            pltpu.VMEM((PBR, RW), jnp.int32),
            pltpu.VMEM((PBR, RW), jnp.int32),
            pltpu.VMEM((PB,), F32),
            pltpu.VMEM((3, PC, D), F32),
            pltpu.VMEM((3, PC, D), F32),
            pltpu.VMEM((3, 48), jnp.int32),
            pltpu.SemaphoreType.DMA((3,)),
            pltpu.SemaphoreType.DMA((3,)),
        ],
    )
    return f(src2, dst2, alpha, h1, rh)


@jax.jit
def kernel(h, edge_index, r_h, W, Wr, att_w, loop_w):
    src = edge_index[0]
    dst = edge_index[1]
    h1, hl, s1, s2, wv = _prep(h, W, Wr, att_w, loop_w)
    er = _er(r_h, wv)
    rh = _rh(r_h, Wr)
    src2 = src.reshape(E // RW, RW)
    dst2 = dst.reshape(E // RW, RW)
    alpha, den = _sc_softmax(src2, dst2, er.reshape(E),
                             s1.reshape(N), s2.reshape(N))
    aggp = _sc_agg(src2, dst2, alpha, h1, rh)
    out = _epi(aggp, hl, h1, den[0].reshape(NP, 1))
    return out


# speedup vs baseline: 1.8395x; 1.8395x over previous
import jax
import jax.numpy as jnp
from jax import lax
from jax.experimental import pallas as pl
from jax.experimental.pallas import tpu as pltpu
from jax.experimental.pallas import tpu_sc as plsc

N = 10000
E = 320000
D = 128

NC = 2
NS = 16
NW = NC * NS
EC = E // NW
RW = 80
ROWS_W = EC // RW
NP = 10240
NPT = NP // NS
PB = 2000
PBR = PB // RW
PBV = PB // 16
F32 = jnp.float32


def _prep_body(h_ref, w_ref, att_ref, loop_ref, h1_ref, hl_ref, s1_ref, s2_ref):
    h1 = lax.dot_general(h_ref[...], w_ref[...], (((1,), (1,)), ((), ())),
                         preferred_element_type=F32)
    h1_ref[...] = h1
    hl_ref[...] = jnp.dot(h1, loop_ref[...], preferred_element_type=F32)
    s1_ref[...] = lax.dot_general(h1, att_ref[:, 0:D], (((1,), (1,)), ((), ())),
                                  preferred_element_type=F32)
    s2_ref[...] = lax.dot_general(h1, att_ref[:, D:2 * D], (((1,), (1,)), ((), ())),
                                  preferred_element_type=F32)


def _prep(h, W, att_w, loop_w):
    return pl.pallas_call(
        _prep_body,
        out_shape=[
            jax.ShapeDtypeStruct((N, D), F32),
            jax.ShapeDtypeStruct((N, D), F32),
            jax.ShapeDtypeStruct((N, 1), F32),
            jax.ShapeDtypeStruct((N, 1), F32),
        ],
    )(h, W, att_w, loop_w)


_BE = 4000


def _rh_body(x_ref, wr_ref, att_ref, rh_ref, er_ref):
    rh = lax.dot_general(x_ref[...], wr_ref[...], (((1,), (1,)), ((), ())),
                         preferred_element_type=F32)
    rh_ref[...] = rh
    er_ref[...] = lax.dot_general(rh, att_ref[:, 2 * D:3 * D],
                                  (((1,), (1,)), ((), ())),
                                  preferred_element_type=F32)


def _rh(r_h, Wr, att_w):
    return pl.pallas_call(
        _rh_body,
        grid=(E // _BE,),
        in_specs=[
            pl.BlockSpec((_BE, D), lambda i: (i, 0)),
            pl.BlockSpec((D, D), lambda i: (0, 0)),
            pl.BlockSpec((1, 3 * D), lambda i: (0, 0)),
        ],
        out_specs=[
            pl.BlockSpec((_BE, D), lambda i: (i, 0)),
            pl.BlockSpec((_BE, 1), lambda i: (i, 0)),
        ],
        out_shape=[
            jax.ShapeDtypeStruct((E, D), F32),
            jax.ShapeDtypeStruct((E, 1), F32),
        ],
    )(r_h, Wr, att_w)


_BN = 2000


def _epi_body(aggp_ref, hl_ref, h1_ref, den_ref, o_ref):
    h_new = aggp_ref[0] + aggp_ref[1] + hl_ref[...]
    sel = jnp.where(den_ref[...] > 0, h_new, h1_ref[...])
    o_ref[...] = jnp.maximum(sel, 0.0)


def _epi(aggp, hl, h1, den):
    return pl.pallas_call(
        _epi_body,
        grid=(N // _BN,),
        in_specs=[
            pl.BlockSpec((2, _BN, D), lambda i: (0, i, 0)),
            pl.BlockSpec((_BN, D), lambda i: (i, 0)),
            pl.BlockSpec((_BN, D), lambda i: (i, 0)),
            pl.BlockSpec((_BN, 1), lambda i: (i, 0)),
        ],
        out_specs=pl.BlockSpec((_BN, D), lambda i: (i, 0)),
        out_shape=jax.ShapeDtypeStruct((N, D), F32),
    )(aggp, hl, h1, den)


def _sc_body(src2_h, dst2_h, er_h, s1_h, s2_h, h1_h, rh_h,
             agg_out, den_out,
             exo, srcb2, dstb2, erb, srcq, dstq, alq, mxv, mxm,
             mx_sh, den_sh, agg_sh):
    c = lax.axis_index("c")
    s = lax.axis_index("s")
    wid = c * NS + s
    own = wid * EC
    mir = ((1 - c) * NS + s) * EC
    zv = jnp.zeros((16,), F32)

    def zden_body(i, _):
        erb[pl.ds(i * 16, 16)] = zv
        return 0
    lax.fori_loop(0, NPT // 16, zden_body, 0)
    pltpu.sync_copy(erb.at[pl.ds(0, NPT)], den_sh.at[pl.ds(s * NPT, NPT)])

    def bphase(s1v, s2v, exm):
        pltpu.sync_copy(s1_h, s1v)
        pltpu.sync_copy(s2_h, s2v)
        mx0 = jnp.full((16,), -jnp.inf, F32)

        def b1_chunk(base, ebuf, mx_in):
            def piece(p, mx):
                off = base + p * PB
                pltpu.sync_copy(src2_h.at[pl.ds(off // RW, PBR), :], srcb2)
                pltpu.sync_copy(dst2_h.at[pl.ds(off // RW, PBR), :], dstb2)
                pltpu.sync_copy(er_h.at[pl.ds(off, PB)], erb)

                def row(j, m):
                    for k in range(RW // 16):
                        sl = pl.ds(k * 16, 16)
                        z = (plsc.load_gather(s1v, [srcb2[j, sl]])
                             + plsc.load_gather(s2v, [dstb2[j, sl]])
                             + erb[pl.ds(j * RW + k * 16, 16)])
                        e = jnp.where(z >= 0, z, z * F32(0.01))
                        ebuf[pl.ds(p * PB + j * RW + k * 16, 16)] = e
                        m = jnp.maximum(m, e)
                    return m
                return lax.fori_loop(0, PBR, row, mx)
            return lax.fori_loop(0, EC // PB, piece, mx_in)

        mx = b1_chunk(own, exo, mx0)
        mx = b1_chunk(mir, exm, mx)

        mxv[...] = mx
        pltpu.sync_copy(mxv, mx_sh.at[s])
        plsc.subcore_barrier()
        pltpu.sync_copy(mx_sh, mxm)
        cm = mxm[0, :]
        for i in range(1, NS):
            cm = jnp.maximum(cm, mxm[i, :])
        gmax = jnp.max(cm)

        def b2_chunk(base, ebuf):
            def piece(p, _):
                off = base + p * PB
                pltpu.sync_copy(dst2_h.at[pl.ds(off // RW, PBR), :],
                                dstb2)

                def vec(i, __):
                    sl = pl.ds(p * PB + i * 16, 16)
                    ebuf[sl] = jnp.exp(ebuf[sl] - gmax)
                    return 0
                lax.fori_loop(0, PBV, vec, 0)

                def sc2(j, __):
                    pltpu.sync_copy(ebuf.at[pl.ds(p * PB + j * RW, RW)],
                                    den_sh.at[dstb2.at[j]], add=True)
                    return 0
                lax.fori_loop(0, PBR, sc2, 0)
                return 0
            lax.fori_loop(0, EC // PB, piece, 0)

        b2_chunk(own, exo)
        b2_chunk(mir, exm)

    pl.run_scoped(bphase,
                  pltpu.VMEM((N,), F32),
                  pltpu.VMEM((N,), F32),
                  pltpu.VMEM((EC,), F32))
    plsc.subcore_barrier()

    def foldphase(rdv):
        pltpu.sync_copy(den_sh, rdv)

        def rd_body(i, _):
            sl = pl.ds(i * 16, 16)
            v = rdv[sl]
            rdv[sl] = jnp.where(v > 0, F32(1.0) / v, F32(1.0))
            return 0
        lax.fori_loop(0, NP // 16, rd_body, 0)
        pltpu.sync_copy(den_sh.at[pl.ds(s * NPT, NPT)],
                        den_out.at[c, pl.ds(s * NPT, NPT)])

        def fold_piece(p, _):
            off = own + p * PB
            pltpu.sync_copy(dst2_h.at[pl.ds(off // RW, PBR), :], dstb2)

            def row(j, __):
                for k in range(RW // 16):
                    rd = plsc.load_gather(rdv, [dstb2[j, pl.ds(k * 16, 16)]])
                    sl = pl.ds(p * PB + j * RW + k * 16, 16)
                    exo[sl] = exo[sl] * rd
                return 0
            lax.fori_loop(0, PBR, row, 0)
            return 0
        lax.fori_loop(0, EC // PB, fold_piece, 0)

    pl.run_scoped(foldphase, pltpu.VMEM((NP,), F32))

    def cphase(h1q, rhq):
        def zrow_body(i, _):
            for kk in range(8):
                h1q[i, pl.ds(kk * 16, 16)] = zv
            return 0
        lax.fori_loop(0, RW, zrow_body, 0)

        def zagg_body(j, _):
            pltpu.sync_copy(h1q, agg_sh.at[pl.ds(s * NPT + j * RW, RW), :])
            return 0
        lax.fori_loop(0, NPT // RW, zagg_body, 0)
        plsc.subcore_barrier()

        rowbase = wid * ROWS_W

        def cpiece(r, _):
            erow = own + r * RW
            pltpu.sync_copy(src2_h.at[rowbase + r], srcq)
            pltpu.sync_copy(dst2_h.at[rowbase + r], dstq)
            pltpu.sync_copy(rh_h.at[pl.ds(erow, RW), :], rhq)
            pltpu.sync_copy(h1_h.at[srcq], h1q)

            def edge_body(i, __):
                a = exo[pl.ds(r * RW + i, 16)][0]
                for kk in range(8):
                    sl = pl.ds(kk * 16, 16)
                    rhq[i, sl] = a * (h1q[i, sl] + rhq[i, sl])
                return 0
            lax.fori_loop(0, RW, edge_body, 0)

            pltpu.sync_copy(rhq, agg_sh.at[dstq], add=True)
            return 0
        lax.fori_loop(0, ROWS_W, cpiece, 0)

        plsc.subcore_barrier()
        pltpu.sync_copy(agg_sh.at[pl.ds(s * NPT, NPT), :],
                        agg_out.at[c, pl.ds(s * NPT, NPT), :])

    pl.run_scoped(cphase,
                  pltpu.VMEM((RW, D), F32),
                  pltpu.VMEM((RW, D), F32))


def _sc_edges(src2, dst2, er, s1, s2, h1, rh):
    mesh = plsc.VectorSubcoreMesh(core_axis_name="c", subcore_axis_name="s")
    f = pl.kernel(
        _sc_body, mesh=mesh,
        compiler_params=pltpu.CompilerParams(use_tc_tiling_on_sc=False,
                                             needs_layout_passes=False),
        out_type=[
            jax.ShapeDtypeStruct((NC, NP, D), F32),
            jax.ShapeDtypeStruct((NC, NP), F32),
        ],
        scratch_types=[
            pltpu.VMEM((EC + 16,), F32),
            pltpu.VMEM((PBR, RW), jnp.int32),
            pltpu.VMEM((PBR, RW), jnp.int32),
            pltpu.VMEM((PB,), F32),
            pltpu.VMEM((RW,), jnp.int32),
            pltpu.VMEM((RW,), jnp.int32),
            pltpu.VMEM((RW + 16,), F32),
            pltpu.VMEM((16,), F32),
            pltpu.VMEM((NS, 16), F32),
            pltpu.VMEM_SHARED((NS, 16), F32),
            pltpu.VMEM_SHARED((NP,), F32),
            pltpu.VMEM_SHARED((NP, D), F32),
        ],
    )
    return f(src2, dst2, er, s1, s2, h1, rh)


@jax.jit
def kernel(h, edge_index, r_h, W, Wr, att_w, loop_w):
    src = edge_index[0]
    dst = edge_index[1]
    h1, hl, s1, s2 = _prep(h, W, att_w, loop_w)
    rh, er = _rh(r_h, Wr, att_w)
    src2 = src.reshape(E // RW, RW)
    dst2 = dst.reshape(E // RW, RW)
    aggp, den = _sc_edges(src2, dst2, er.reshape(E),
                          s1.reshape(N), s2.reshape(N), h1, rh)
    out = _epi(aggp, hl, h1, den[0].reshape(NP, 1))
    return out
